# baseline (speedup 1.0000x reference)
import functools

import jax
import jax.numpy as jnp
from jax import lax
from jax.experimental import pallas as pl
from jax.experimental.pallas import tpu as pltpu

B, N, S, K, D = 16, 2048, 512, 24, 128


def _rtne_bf16(x):
    b = lax.bitcast_convert_type(x, jnp.uint32)
    r = (b + 0x7FFF + ((b >> 16) & 1)) & jnp.uint32(0xFFFF0000)
    return lax.bitcast_convert_type(r, jnp.float32)


def _fps_kernel(xyzT_ref, idx_ref, xyzs_ref):
    X = xyzT_ref[0]
    Y = xyzT_ref[1]
    Z = xyzT_ref[2]
    lane = lax.broadcasted_iota(jnp.int32, (B, N), 1)
    lane_s = lax.broadcasted_iota(jnp.int32, (B, S), 1)
    row_s = lax.broadcasted_iota(jnp.int32, (B, S), 0)
    key = row_s * 1024 + lane_s

    def body(i, st):
        dist, far, idxacc, sx, sy, sz = st
        m = lane == far
        cx = jnp.sum(jnp.where(m, X, 0.0), axis=1, keepdims=True)
        cy = jnp.sum(jnp.where(m, Y, 0.0), axis=1, keepdims=True)
        cz = jnp.sum(jnp.where(m, Z, 0.0), axis=1, keepdims=True)
        sel = key == row_s * 1024 + i
        idxacc = jnp.where(sel, jnp.broadcast_to(far, (B, S)), idxacc)
        sx = jnp.where(sel, jnp.broadcast_to(cx, (B, S)), sx)
        sy = jnp.where(sel, jnp.broadcast_to(cy, (B, S)), sy)
        sz = jnp.where(sel, jnp.broadcast_to(cz, (B, S)), sz)
        dx = X - cx
        dy = Y - cy
        dz = Z - cz
        d = dx * dx + dy * dy + dz * dz
        dist = jnp.minimum(dist, d)
        mx = jnp.max(dist, axis=1, keepdims=True)
        far = jnp.min(jnp.where(dist == mx, lane, N), axis=1, keepdims=True)
        return dist, far, idxacc, sx, sy, sz

    init = (jnp.full((B, N), 1e10, jnp.float32),
            jnp.zeros((B, 1), jnp.int32),
            jnp.zeros((B, S), jnp.int32),
            jnp.zeros((B, S), jnp.float32),
            jnp.zeros((B, S), jnp.float32),
            jnp.zeros((B, S), jnp.float32))
    _, _, idxacc, sx, sy, sz = lax.fori_loop(0, S, body, init)
    idx_ref[...] = idxacc
    xyzs_ref[0] = sx
    xyzs_ref[1] = sy
    xyzs_ref[2] = sz


def _run_fps(xyzT):
    return pl.pallas_call(
        _fps_kernel,
        out_shape=[jax.ShapeDtypeStruct((B, S), jnp.int32),
                   jax.ShapeDtypeStruct((3, B, S), jnp.float32)],
        in_specs=[pl.BlockSpec((3, B, N), lambda: (0, 0, 0))],
        out_specs=[pl.BlockSpec((B, S), lambda: (0, 0)),
                   pl.BlockSpec((3, B, S), lambda: (0, 0, 0))],
    )(xyzT)


_RT = 128


def _knn_kernel(xyzT_ref, xyzs_ref, qn_ref, idx_ref):
    p = xyzT_ref[0]
    X = p[0:1, :]
    Y = p[1:2, :]
    Z = p[2:3, :]
    q = xyzs_ref[0]
    qx = q[:, 0:1]
    qy = q[:, 1:2]
    qz = q[:, 2:3]
    pn = X * X + Y * Y + Z * Z
    qn = qn_ref[0]
    dot = (_rtne_bf16(qx) * _rtne_bf16(X)
           + _rtne_bf16(qy) * _rtne_bf16(Y)
           + _rtne_bf16(qz) * _rtne_bf16(Z))
    Dm = (qn + pn) - 2.0 * dot
    lane = lax.broadcasted_iota(jnp.int32, (_RT, N), 1)
    k_iota = lax.broadcasted_iota(jnp.int32, (_RT, K), 1)

    def body(k, st):
        Dc, acc = st
        mn = jnp.min(Dc, axis=1, keepdims=True)
        cand = jnp.where(Dc <= mn, lane, N)
        j = jnp.min(cand, axis=1, keepdims=True)
        acc = jnp.where(k_iota == k, j, acc)
        Dc = jnp.where(lane == j, jnp.float32(1e30), Dc)
        return Dc, acc

    _, acc = lax.fori_loop(0, K, body, (Dm, jnp.zeros((_RT, K), jnp.int32)))
    idx_ref[0] = acc


def _run_knn(xyzTb, xyz_sampled, qn):
    grid = (B, S // _RT)
    return pl.pallas_call(
        _knn_kernel,
        grid=grid,
        out_shape=jax.ShapeDtypeStruct((B, S, K), jnp.int32),
        in_specs=[pl.BlockSpec((1, 3, N), lambda b, t: (b, 0, 0)),
                  pl.BlockSpec((1, _RT, 3), lambda b, t: (b, t, 0)),
                  pl.BlockSpec((1, _RT, 1), lambda b, t: (b, t, 0))],
        out_specs=pl.BlockSpec((1, _RT, K), lambda b, t: (b, t, 0)),
    )(xyzTb, xyz_sampled, qn)


from jax.experimental.pallas import tpu_sc as plsc

_SB = 256
_G = 4
_NG = _SB // _G
_MF = B * S * K * D
_MX = B * S * K * 3


def _sc_gather(feat2d, xyz128, idxk, fpsk):
    mesh = plsc.VectorSubcoreMesh(core_axis_name="c", subcore_axis_name="s")

    @functools.partial(
        pl.kernel, mesh=mesh,
        out_type=[jax.ShapeDtypeStruct((B * S * K * D,), jnp.float32),
                  jax.ShapeDtypeStruct((B * S, D), jnp.float32),
                  jax.ShapeDtypeStruct((B * S * K, 16), jnp.float32),
                  jax.ShapeDtypeStruct((32, 4, 16), jnp.float32)],
        scratch_types=[pltpu.VMEM((_SB * K,), jnp.int32),
                       pltpu.VMEM((_SB,), jnp.int32),
                       pltpu.VMEM((_SB, D), jnp.float32),
                       pltpu.VMEM((_SB, D), jnp.float32),
                       pltpu.VMEM((_G * K, D), jnp.float32),
                       pltpu.VMEM((_G * K, D), jnp.float32),
                       pltpu.VMEM((_G * K * D,), jnp.float32),
                       pltpu.VMEM((_G * K, 16), jnp.float32),
                       pltpu.VMEM((4, 16), jnp.float32),
                       pltpu.SemaphoreType.DMA,
                       pltpu.SemaphoreType.DMA],
    )
    def k(feat_hbm, xyz_hbm, idxk_hbm, fps_hbm,
          fdiff_hbm, fsamp_hbm, xdiff_hbm, stats_hbm,
          idxg_v, fpsg_v, cent_v, xcent_v, rows_v, xrow_v, dst_v, xdst_v,
          stat_v, sem, sem2):
        wid = lax.axis_index("s") * 2 + lax.axis_index("c")
        base = wid * _SB
        b = wid // 2
        boff = b * N

        pltpu.sync_copy(idxk_hbm.at[pl.ds(base * K, _SB * K)], idxg_v)
        pltpu.sync_copy(fps_hbm.at[pl.ds(base, _SB)], fpsg_v)

        def to_global(i, _):
            v = fpsg_v[pl.ds(i * 16, 16)]
            fpsg_v[pl.ds(i * 16, 16)] = v + boff
            return 0
        lax.fori_loop(0, _SB // 16, to_global, 0)

        def to_global2(i, _):
            v = idxg_v[pl.ds(i * 16, 16)]
            idxg_v[pl.ds(i * 16, 16)] = v + boff
            return 0
        lax.fori_loop(0, _SB * K // 16, to_global2, 0)

        pltpu.async_copy(feat_hbm.at[fpsg_v.at[pl.ds(0, 128)]],
                         cent_v.at[pl.ds(0, 128)], sem).wait()
        pltpu.async_copy(feat_hbm.at[fpsg_v.at[pl.ds(128, 128)]],
                         cent_v.at[pl.ds(128, 128)], sem).wait()
        pltpu.async_copy(xyz_hbm.at[fpsg_v.at[pl.ds(0, 128)]],
                         xcent_v.at[pl.ds(0, 128)], sem).wait()
        pltpu.async_copy(xyz_hbm.at[fpsg_v.at[pl.ds(128, 128)]],
                         xcent_v.at[pl.ds(128, 128)], sem).wait()
        pltpu.sync_copy(cent_v, fsamp_hbm.at[pl.ds(base, _SB)])

        def group(g, acc):
            acc_s, acc_q, xac_s, xac_q = acc
            pltpu.async_copy(
                feat_hbm.at[idxg_v.at[pl.ds(g * (_G * K), _G * K)]],
                rows_v, sem)
            pltpu.async_copy(
                xyz_hbm.at[idxg_v.at[pl.ds(g * (_G * K), _G * K)]],
                xrow_v, sem2)
            pltpu.make_async_copy(
                feat_hbm.at[idxg_v.at[pl.ds(g * (_G * K), _G * K)]],
                rows_v, sem).wait()
            pltpu.make_async_copy(
                xyz_hbm.at[idxg_v.at[pl.ds(g * (_G * K), _G * K)]],
                xrow_v, sem2).wait()

            def row(r, acc2):
                a_s, a_q, x_s, x_q = acc2
                t = r // K
                samp = g * _G + t
                rbase = r * D
                for c in range(D // 16):
                    ct = cent_v[samp, pl.ds(c * 16, 16)]
                    rv = rows_v[r, pl.ds(c * 16, 16)]
                    dv = rv - ct
                    dst_v[pl.ds(rbase + c * 16, 16)] = dv
                    a_s = a_s + dv
                    a_q = a_q + dv * dv
                xd = (xrow_v[r, pl.ds(0, 16)]
                      - xcent_v[samp, pl.ds(0, 16)])
                xdst_v[r, pl.ds(0, 16)] = xd
                x_s = x_s + xd
                x_q = x_q + xd * xd
                return a_s, a_q, x_s, x_q

            acc = lax.fori_loop(0, _G * K, row, (acc_s, acc_q, xac_s, xac_q))
            pltpu.sync_copy(dst_v,
                            fdiff_hbm.at[pl.ds((base + g * _G) * K * D,
                                               _G * K * D)])
            pltpu.sync_copy(xdst_v,
                            xdiff_hbm.at[pl.ds((base + g * _G) * K, _G * K)])
            return acc

        zero = jnp.zeros((16,), jnp.float32)
        acc_s, acc_q, xac_s, xac_q = lax.fori_loop(
            0, _NG, group, (zero, zero, zero, zero))

        stat_v[0] = acc_s
        stat_v[1] = acc_q
        stat_v[2] = xac_s
        stat_v[3] = xac_q
        pltpu.sync_copy(stat_v, stats_hbm.at[wid])

    return k(feat2d, xyz128, idxk, fpsk)


_ST = 64


def _fin_kernel(fdiff_ref, fsamp_ref, xdiff_ref, stats_ref, fout_ref,
                xout_ref):
    st = stats_ref[...]
    fs = jnp.sum(st[:, 0, :])
    fq = jnp.sum(st[:, 1, :])
    xs = jnp.sum(st[:, 2, :])
    xq = jnp.sum(st[:, 3, :])
    fstd = jnp.sqrt((fq - fs * fs / _MF) / (_MF - 1))
    xstd = jnp.sqrt((xq - xs * xs / _MX) / (_MX - 1))
    fout_ref[0, :, :, 0:D] = fdiff_ref[0] / (fstd + 1e-05)
    rep = jnp.broadcast_to(fsamp_ref[0][:, None, :], (_ST, K, D))
    fout_ref[0, :, :, D:2 * D] = rep
    xout_ref[0] = xdiff_ref[0][:, :, 0:3] / (xstd + 1e-05)


def _run_finalize(fdiff, fsamp, xdiff, stats):
    grid = (B, S // _ST)
    return pl.pallas_call(
        _fin_kernel,
        grid=grid,
        out_shape=[jax.ShapeDtypeStruct((B, S, K, 2 * D), jnp.float32),
                   jax.ShapeDtypeStruct((B, S, K, 3), jnp.float32)],
        in_specs=[pl.BlockSpec((1, _ST, K, D), lambda b, t: (b, t, 0, 0)),
                  pl.BlockSpec((1, _ST, D), lambda b, t: (b, t, 0)),
                  pl.BlockSpec((1, _ST, K, 16), lambda b, t: (b, t, 0, 0)),
                  pl.BlockSpec((32, 4, 16), lambda b, t: (0, 0, 0))],
        out_specs=[pl.BlockSpec((1, _ST, K, 2 * D), lambda b, t: (b, t, 0, 0)),
                   pl.BlockSpec((1, _ST, K, 3), lambda b, t: (b, t, 0, 0))],
    )(fdiff, fsamp, xdiff, stats)


def kernel(xyz, feat):
    xyzT = jnp.transpose(xyz, (2, 0, 1))
    fps_idx, xyz_sT = _run_fps(xyzT)
    xyz_sampled = jnp.transpose(xyz_sT, (1, 2, 0))
    xyzTb = jnp.transpose(xyz, (0, 2, 1))
    qn = jnp.sum(xyz_sampled ** 2, axis=-1)[..., None]
    idx_knn = _run_knn(xyzTb, xyz_sampled, qn)

    feat2d = feat.reshape(B * N, D)
    xyz128 = jnp.pad(xyz, ((0, 0), (0, 0), (0, D - 3))).reshape(B * N, D)
    idxk = idx_knn.reshape(-1)
    fpsk = fps_idx.reshape(-1)
    fdiff, fsamp, xdiff, stats = _sc_gather(feat2d, xyz128, idxk, fpsk)
    feat_sampled = fsamp.reshape(B, S, D)
    fdiff = fdiff.reshape(B, S, K, D)
    xdiff = xdiff.reshape(B, S, K, 16)
    feat_knn, xyz_knn = _run_finalize(fdiff, feat_sampled, xdiff, stats)
    return (xyz_sampled, feat_sampled, xyz_knn, feat_knn)
